# baseline (speedup 1.0000x reference)
import functools

import jax
import jax.numpy as jnp
from jax import lax
from jax.experimental import pallas as pl
from jax.experimental.pallas import tpu as pltpu, tpu_sc as plsc

RANK = 4
RES = 512
NC = 2
NS = 16
L = 16
NW = NC * NS

CH = 512
NIDX = 3 * CH
IPT = 128
NDMA = NIDX // IPT
NREC = 3 * RES * RES


def _tri_body(cx_hbm, cy_hbm, cz_hbm, tab_hbm, out_hbm,
              cv0, cv1, idx0, idx1, w0, w1, rows0, rows1, ov,
              sem_c, sem_g, *, n_pts):
    np_w = n_pts // NW
    nchunk = np_w // CH
    wid = lax.axis_index("s") * NC + lax.axis_index("c")
    wbase = wid * np_w

    def fire_coords(chunk, cv):
        base = wbase + chunk * CH
        pltpu.async_copy(cx_hbm.at[pl.ds(base, CH)], cv.at[pl.ds(0, CH)], sem_c)
        pltpu.async_copy(cy_hbm.at[pl.ds(base, CH)], cv.at[pl.ds(CH, CH)], sem_c)
        pltpu.async_copy(cz_hbm.at[pl.ds(base, CH)], cv.at[pl.ds(2 * CH, CH)], sem_c)

    def wait_coords(cv):
        for a in range(3):
            pltpu.make_async_copy(cx_hbm.at[pl.ds(0, CH)],
                                  cv.at[pl.ds(a * CH, CH)], sem_c).wait()

    def phase_b(cv, idxv, wv):
        def idx_body(i, c2):
            o = i * L
            cx = cv[pl.ds(o, L)]
            cy = cv[pl.ds(CH + o, L)]
            cz = cv[pl.ds(2 * CH + o, L)]
            for ci, (ax, ay) in enumerate(((cx, cy), (cx, cz), (cy, cz))):
                x0 = jnp.clip(ax.astype(jnp.int32), 0, RES - 2)
                y0 = jnp.clip(ay.astype(jnp.int32), 0, RES - 2)
                fx = ax - x0.astype(jnp.float32)
                fy = ay - y0.astype(jnp.float32)
                idxv[pl.ds(ci * CH + o, L)] = y0 * RES + x0 + ci * (RES * RES)
                wv[pl.ds((2 * ci) * CH + o, L)] = fx
                wv[pl.ds((2 * ci + 1) * CH + o, L)] = fy
            return c2

        lax.fori_loop(0, CH // L, idx_body, 0)

    def fire_gathers(idxv, rowsv):
        for j in range(NDMA):
            pltpu.async_copy(tab_hbm.at[idxv.at[pl.ds(j * IPT, IPT)]],
                             rowsv.at[pl.ds(j * IPT, IPT)], sem_g)

    def drain_gathers(idxv, rowsv):
        for j in range(NDMA):
            pltpu.make_async_copy(tab_hbm.at[idxv.at[pl.ds(j * IPT, IPT)]],
                                  rowsv.at[pl.ds(j * IPT, IPT)], sem_g).wait()

    def phase_c(chunk, wv, rowsv):
        def grp_body(i, c2):
            o = i * L
            rb = lax.iota(jnp.int32, L) + o
            accs = [None] * RANK
            for ci in range(3):
                fx = wv[pl.ds((2 * ci) * CH + o, L)]
                fy = wv[pl.ds((2 * ci + 1) * CH + o, L)]
                wx0 = 1.0 - fx
                wy0 = 1.0 - fy
                rbp = rb + ci * CH
                for c in range(RANK):
                    v00 = plsc.load_gather(rowsv, [rbp, jnp.full((L,), c, jnp.int32)])
                    v01 = plsc.load_gather(rowsv, [rbp, jnp.full((L,), 4 + c, jnp.int32)])
                    v10 = plsc.load_gather(rowsv, [rbp, jnp.full((L,), 8 + c, jnp.int32)])
                    v11 = plsc.load_gather(rowsv, [rbp, jnp.full((L,), 12 + c, jnp.int32)])
                    val = (v00 * wx0 + v01 * fx) * wy0 + (v10 * wx0 + v11 * fx) * fy
                    accs[c] = val if ci == 0 else accs[c] * val
            s = (accs[0] + accs[1]) + (accs[2] + accs[3])
            ov[pl.ds(o, L)] = jnp.maximum(s * 0.25, 0.0)
            return c2

        lax.fori_loop(0, CH // L, grp_body, 0)
        pltpu.sync_copy(ov, out_hbm.at[pl.ds(wbase + chunk * CH, CH)])

    fire_coords(0, cv0)

    def pair_body(g2, carry):
        ga = 2 * g2
        wait_coords(cv0)
        phase_b(cv0, idx0, w0)

        @pl.when(g2 > 0)
        def _():
            drain_gathers(idx1, rows1)

        fire_gathers(idx0, rows0)
        fire_coords(ga + 1, cv1)

        @pl.when(g2 > 0)
        def _():
            phase_c(ga - 1, w1, rows1)

        wait_coords(cv1)
        phase_b(cv1, idx1, w1)
        drain_gathers(idx0, rows0)
        fire_gathers(idx1, rows1)
        fire_coords(jnp.minimum(ga + 2, nchunk - 1), cv0)
        phase_c(ga, w0, rows0)
        return carry

    lax.fori_loop(0, nchunk // 2, pair_body, 0)

    wait_coords(cv0)
    drain_gathers(idx1, rows1)
    phase_c(nchunk - 1, w1, rows1)


def _quad_table(g):
    t = jnp.transpose(g, (1, 2, 0))
    tx = jnp.concatenate([t[:, 1:], t[:, -1:]], axis=1)
    ty = jnp.concatenate([t[1:], t[-1:]], axis=0)
    txy = jnp.concatenate([ty[:, 1:], ty[:, -1:]], axis=1)
    return jnp.concatenate(
        [a.reshape(RES * RES, RANK) for a in (t, tx, ty, txy)], axis=-1)


def kernel(pts, G0, G1, G2, aabb):
    n_rays, n_samples = pts.shape[:2]
    n_pts = n_rays * n_samples

    lo = aabb[0]
    scale = (RES - 1.0) / (aabb[1] - lo)
    cx = ((pts[:, :, 0] - lo[0]) * scale[0]).reshape(-1)
    cy = ((pts[:, :, 1] - lo[1]) * scale[1]).reshape(-1)
    cz = ((pts[:, :, 2] - lo[2]) * scale[2]).reshape(-1)

    table = jnp.concatenate(
        [_quad_table(G0), _quad_table(G1), _quad_table(G2)], axis=0
    )

    mesh = plsc.VectorSubcoreMesh(core_axis_name="c", subcore_axis_name="s",
                                  num_cores=NC, num_subcores=NS)
    run = pl.kernel(
        functools.partial(_tri_body, n_pts=n_pts),
        out_type=jax.ShapeDtypeStruct((n_pts,), jnp.float32),
        mesh=mesh,
        compiler_params=pltpu.CompilerParams(needs_layout_passes=False,
                                             use_tc_tiling_on_sc=False),
        scratch_types=[
            pltpu.VMEM((CH * 3,), jnp.float32),
            pltpu.VMEM((CH * 3,), jnp.float32),
            pltpu.VMEM((NIDX,), jnp.int32),
            pltpu.VMEM((NIDX,), jnp.int32),
            pltpu.VMEM((6 * CH,), jnp.float32),
            pltpu.VMEM((6 * CH,), jnp.float32),
            pltpu.VMEM((NIDX, 4 * RANK), jnp.float32),
            pltpu.VMEM((NIDX, 4 * RANK), jnp.float32),
            pltpu.VMEM((CH,), jnp.float32),
            pltpu.SemaphoreType.DMA,
            pltpu.SemaphoreType.DMA,
        ],
    )
    out = run(cx, cy, cz, table)
    return out.reshape(n_rays, n_samples, 1)


# speedup vs baseline: 1.1073x; 1.1073x over previous
import functools

import jax
import jax.numpy as jnp
from jax import lax
from jax.experimental import pallas as pl
from jax.experimental.pallas import tpu as pltpu, tpu_sc as plsc

RANK = 4
RES = 512
NC = 2
NS = 16
L = 16
NW = NC * NS

CH = 1024
NIDX = 3 * CH
IPT = 128
NDMA = NIDX // IPT
NREC = 3 * RES * RES


def _tri_body(cx_hbm, cy_hbm, cz_hbm, tab_hbm, out_hbm,
              cv0, cv1, idx0, idx1, w0, w1, rows0, rows1, ov,
              sem_c, sem_g, *, n_pts):
    np_w = n_pts // NW
    nchunk = np_w // CH
    wid = lax.axis_index("s") * NC + lax.axis_index("c")
    wbase = wid * np_w

    def fire_coords(chunk, cv):
        base = wbase + chunk * CH
        pltpu.async_copy(cx_hbm.at[pl.ds(base, CH)], cv.at[pl.ds(0, CH)], sem_c)
        pltpu.async_copy(cy_hbm.at[pl.ds(base, CH)], cv.at[pl.ds(CH, CH)], sem_c)
        pltpu.async_copy(cz_hbm.at[pl.ds(base, CH)], cv.at[pl.ds(2 * CH, CH)], sem_c)

    def wait_coords(cv):
        for a in range(3):
            pltpu.make_async_copy(cx_hbm.at[pl.ds(0, CH)],
                                  cv.at[pl.ds(a * CH, CH)], sem_c).wait()

    def phase_b(cv, idxv, wv):
        def idx_body(i, c2):
            o = i * L
            cx = cv[pl.ds(o, L)]
            cy = cv[pl.ds(CH + o, L)]
            cz = cv[pl.ds(2 * CH + o, L)]
            for ci, (ax, ay) in enumerate(((cx, cy), (cx, cz), (cy, cz))):
                x0 = jnp.clip(ax.astype(jnp.int32), 0, RES - 2)
                y0 = jnp.clip(ay.astype(jnp.int32), 0, RES - 2)
                fx = ax - x0.astype(jnp.float32)
                fy = ay - y0.astype(jnp.float32)
                idxv[pl.ds(ci * CH + o, L)] = y0 * RES + x0 + ci * (RES * RES)
                wv[pl.ds((2 * ci) * CH + o, L)] = fx
                wv[pl.ds((2 * ci + 1) * CH + o, L)] = fy
            return c2

        lax.fori_loop(0, CH // L, idx_body, 0)

    def fire_gathers(idxv, rowsv):
        for j in range(NDMA):
            pltpu.async_copy(tab_hbm.at[idxv.at[pl.ds(j * IPT, IPT)]],
                             rowsv.at[pl.ds(j * IPT, IPT)], sem_g)

    def drain_gathers(idxv, rowsv):
        for j in range(NDMA):
            pltpu.make_async_copy(tab_hbm.at[idxv.at[pl.ds(j * IPT, IPT)]],
                                  rowsv.at[pl.ds(j * IPT, IPT)], sem_g).wait()

    def phase_c(chunk, wv, rowsv):
        def grp_body(i, c2):
            o = i * L
            rb = lax.iota(jnp.int32, L) + o
            accs = [None] * RANK
            for ci in range(3):
                fx = wv[pl.ds((2 * ci) * CH + o, L)]
                fy = wv[pl.ds((2 * ci + 1) * CH + o, L)]
                wx0 = 1.0 - fx
                wy0 = 1.0 - fy
                rbp = rb + ci * CH
                for c in range(RANK):
                    v00 = plsc.load_gather(rowsv, [rbp, jnp.full((L,), c, jnp.int32)])
                    v01 = plsc.load_gather(rowsv, [rbp, jnp.full((L,), 4 + c, jnp.int32)])
                    v10 = plsc.load_gather(rowsv, [rbp, jnp.full((L,), 8 + c, jnp.int32)])
                    v11 = plsc.load_gather(rowsv, [rbp, jnp.full((L,), 12 + c, jnp.int32)])
                    val = (v00 * wx0 + v01 * fx) * wy0 + (v10 * wx0 + v11 * fx) * fy
                    accs[c] = val if ci == 0 else accs[c] * val
            s = (accs[0] + accs[1]) + (accs[2] + accs[3])
            ov[pl.ds(o, L)] = jnp.maximum(s * 0.25, 0.0)
            return c2

        lax.fori_loop(0, CH // L, grp_body, 0)
        pltpu.sync_copy(ov, out_hbm.at[pl.ds(wbase + chunk * CH, CH)])

    fire_coords(0, cv0)

    def pair_body(g2, carry):
        ga = 2 * g2
        wait_coords(cv0)
        phase_b(cv0, idx0, w0)

        @pl.when(g2 > 0)
        def _():
            drain_gathers(idx1, rows1)

        fire_gathers(idx0, rows0)
        fire_coords(ga + 1, cv1)

        @pl.when(g2 > 0)
        def _():
            phase_c(ga - 1, w1, rows1)

        wait_coords(cv1)
        phase_b(cv1, idx1, w1)
        drain_gathers(idx0, rows0)
        fire_gathers(idx1, rows1)
        fire_coords(jnp.minimum(ga + 2, nchunk - 1), cv0)
        phase_c(ga, w0, rows0)
        return carry

    lax.fori_loop(0, nchunk // 2, pair_body, 0)

    wait_coords(cv0)
    drain_gathers(idx1, rows1)
    phase_c(nchunk - 1, w1, rows1)


def _quad_table(g):
    t = jnp.transpose(g, (1, 2, 0))
    tx = jnp.concatenate([t[:, 1:], t[:, -1:]], axis=1)
    ty = jnp.concatenate([t[1:], t[-1:]], axis=0)
    txy = jnp.concatenate([ty[:, 1:], ty[:, -1:]], axis=1)
    return jnp.concatenate([t, tx, ty, txy], axis=-1).reshape(RES * RES, 4 * RANK)


def kernel(pts, G0, G1, G2, aabb):
    n_rays, n_samples = pts.shape[:2]
    n_pts = n_rays * n_samples

    lo = aabb[0]
    scale = (RES - 1.0) / (aabb[1] - lo)
    cx = ((pts[:, :, 0] - lo[0]) * scale[0]).reshape(-1)
    cy = ((pts[:, :, 1] - lo[1]) * scale[1]).reshape(-1)
    cz = ((pts[:, :, 2] - lo[2]) * scale[2]).reshape(-1)

    table = jnp.concatenate(
        [_quad_table(G0), _quad_table(G1), _quad_table(G2)], axis=0
    )

    mesh = plsc.VectorSubcoreMesh(core_axis_name="c", subcore_axis_name="s",
                                  num_cores=NC, num_subcores=NS)
    run = pl.kernel(
        functools.partial(_tri_body, n_pts=n_pts),
        out_type=jax.ShapeDtypeStruct((n_pts,), jnp.float32),
        mesh=mesh,
        compiler_params=pltpu.CompilerParams(needs_layout_passes=False,
                                             use_tc_tiling_on_sc=False),
        scratch_types=[
            pltpu.VMEM((CH * 3,), jnp.float32),
            pltpu.VMEM((CH * 3,), jnp.float32),
            pltpu.VMEM((NIDX,), jnp.int32),
            pltpu.VMEM((NIDX,), jnp.int32),
            pltpu.VMEM((6 * CH,), jnp.float32),
            pltpu.VMEM((6 * CH,), jnp.float32),
            pltpu.VMEM((NIDX, 4 * RANK), jnp.float32),
            pltpu.VMEM((NIDX, 4 * RANK), jnp.float32),
            pltpu.VMEM((CH,), jnp.float32),
            pltpu.SemaphoreType.DMA,
            pltpu.SemaphoreType.DMA,
        ],
    )
    out = run(cx, cy, cz, table)
    return out.reshape(n_rays, n_samples, 1)


# speedup vs baseline: 1.1080x; 1.0007x over previous
import functools

import jax
import jax.numpy as jnp
from jax import lax
from jax.experimental import pallas as pl
from jax.experimental.pallas import tpu as pltpu, tpu_sc as plsc

RANK = 4
RES = 512
NC = 2
NS = 16
L = 16
NW = NC * NS

CH = 512
NIDX = 3 * CH
IPT = 128
NDMA = NIDX // IPT
NREC = 3 * RES * RES


def _tri_body(cx_hbm, cy_hbm, cz_hbm, tab_hbm, out_hbm,
              cv0, cv1, idx0, idx1, w0, w1, rows0, rows1, ov,
              sem_c, sem_g, *, n_pts):
    np_w = n_pts // NW
    nchunk = np_w // CH
    wid = lax.axis_index("s") * NC + lax.axis_index("c")
    wbase = wid * np_w

    def fire_coords(chunk, cv):
        base = wbase + chunk * CH
        pltpu.async_copy(cx_hbm.at[pl.ds(base, CH)], cv.at[pl.ds(0, CH)], sem_c)
        pltpu.async_copy(cy_hbm.at[pl.ds(base, CH)], cv.at[pl.ds(CH, CH)], sem_c)
        pltpu.async_copy(cz_hbm.at[pl.ds(base, CH)], cv.at[pl.ds(2 * CH, CH)], sem_c)

    def wait_coords(cv):
        for a in range(3):
            pltpu.make_async_copy(cx_hbm.at[pl.ds(0, CH)],
                                  cv.at[pl.ds(a * CH, CH)], sem_c).wait()

    def phase_b(cv, idxv, wv):
        def idx_body(i, c2):
            o = i * L
            cx = cv[pl.ds(o, L)]
            cy = cv[pl.ds(CH + o, L)]
            cz = cv[pl.ds(2 * CH + o, L)]
            for ci, (ax, ay) in enumerate(((cx, cy), (cx, cz), (cy, cz))):
                x0 = jnp.clip(ax.astype(jnp.int32), 0, RES - 2)
                y0 = jnp.clip(ay.astype(jnp.int32), 0, RES - 2)
                fx = ax - x0.astype(jnp.float32)
                fy = ay - y0.astype(jnp.float32)
                idxv[pl.ds(ci * CH + o, L)] = y0 * RES + x0 + ci * (RES * RES)
                wv[pl.ds((2 * ci) * CH + o, L)] = fx
                wv[pl.ds((2 * ci + 1) * CH + o, L)] = fy
            return c2

        lax.fori_loop(0, CH // L, idx_body, 0)

    def fire_gathers(idxv, rowsv):
        for j in range(NDMA):
            pltpu.async_copy(tab_hbm.at[idxv.at[pl.ds(j * IPT, IPT)]],
                             rowsv.at[pl.ds(j * IPT, IPT)], sem_g)

    def drain_gathers(idxv, rowsv):
        for j in range(NDMA):
            pltpu.make_async_copy(tab_hbm.at[idxv.at[pl.ds(j * IPT, IPT)]],
                                  rowsv.at[pl.ds(j * IPT, IPT)], sem_g).wait()

    def phase_c(chunk, wv, rowsv):
        def grp_body(i, c2):
            o = i * L
            rb = lax.iota(jnp.int32, L) + o
            accs = [None] * RANK
            for ci in range(3):
                fx = wv[pl.ds((2 * ci) * CH + o, L)]
                fy = wv[pl.ds((2 * ci + 1) * CH + o, L)]
                wx0 = 1.0 - fx
                wy0 = 1.0 - fy
                rbp = rb + ci * CH
                for c in range(RANK):
                    v00 = plsc.load_gather(rowsv, [rbp, jnp.full((L,), c, jnp.int32)])
                    v01 = plsc.load_gather(rowsv, [rbp, jnp.full((L,), 4 + c, jnp.int32)])
                    v10 = plsc.load_gather(rowsv, [rbp, jnp.full((L,), 8 + c, jnp.int32)])
                    v11 = plsc.load_gather(rowsv, [rbp, jnp.full((L,), 12 + c, jnp.int32)])
                    val = (v00 * wx0 + v01 * fx) * wy0 + (v10 * wx0 + v11 * fx) * fy
                    accs[c] = val if ci == 0 else accs[c] * val
            s = (accs[0] + accs[1]) + (accs[2] + accs[3])
            ov[pl.ds(o, L)] = jnp.maximum(s * 0.25, 0.0)
            return c2

        lax.fori_loop(0, CH // L, grp_body, 0)
        pltpu.sync_copy(ov, out_hbm.at[pl.ds(wbase + chunk * CH, CH)])

    fire_coords(0, cv0)

    def pair_body(g2, carry):
        ga = 2 * g2
        wait_coords(cv0)
        phase_b(cv0, idx0, w0)

        @pl.when(g2 > 0)
        def _():
            drain_gathers(idx1, rows1)

        fire_gathers(idx0, rows0)
        fire_coords(ga + 1, cv1)

        @pl.when(g2 > 0)
        def _():
            phase_c(ga - 1, w1, rows1)

        wait_coords(cv1)
        phase_b(cv1, idx1, w1)
        drain_gathers(idx0, rows0)
        fire_gathers(idx1, rows1)
        fire_coords(jnp.minimum(ga + 2, nchunk - 1), cv0)
        phase_c(ga, w0, rows0)
        return carry

    lax.fori_loop(0, nchunk // 2, pair_body, 0)

    wait_coords(cv0)
    drain_gathers(idx1, rows1)
    phase_c(nchunk - 1, w1, rows1)


def _quad_table(g):
    t = jnp.transpose(g, (1, 2, 0))
    tx = jnp.concatenate([t[:, 1:], t[:, -1:]], axis=1)
    ty = jnp.concatenate([t[1:], t[-1:]], axis=0)
    txy = jnp.concatenate([ty[:, 1:], ty[:, -1:]], axis=1)
    return jnp.concatenate([t, tx, ty, txy], axis=-1).reshape(RES * RES, 4 * RANK)


def kernel(pts, G0, G1, G2, aabb):
    n_rays, n_samples = pts.shape[:2]
    n_pts = n_rays * n_samples

    lo = aabb[0]
    scale = (RES - 1.0) / (aabb[1] - lo)
    cx = ((pts[:, :, 0] - lo[0]) * scale[0]).reshape(-1)
    cy = ((pts[:, :, 1] - lo[1]) * scale[1]).reshape(-1)
    cz = ((pts[:, :, 2] - lo[2]) * scale[2]).reshape(-1)

    table = jnp.concatenate(
        [_quad_table(G0), _quad_table(G1), _quad_table(G2)], axis=0
    )

    mesh = plsc.VectorSubcoreMesh(core_axis_name="c", subcore_axis_name="s",
                                  num_cores=NC, num_subcores=NS)
    run = pl.kernel(
        functools.partial(_tri_body, n_pts=n_pts),
        out_type=jax.ShapeDtypeStruct((n_pts,), jnp.float32),
        mesh=mesh,
        compiler_params=pltpu.CompilerParams(needs_layout_passes=False,
                                             use_tc_tiling_on_sc=False),
        scratch_types=[
            pltpu.VMEM((CH * 3,), jnp.float32),
            pltpu.VMEM((CH * 3,), jnp.float32),
            pltpu.VMEM((NIDX,), jnp.int32),
            pltpu.VMEM((NIDX,), jnp.int32),
            pltpu.VMEM((6 * CH,), jnp.float32),
            pltpu.VMEM((6 * CH,), jnp.float32),
            pltpu.VMEM((NIDX, 4 * RANK), jnp.float32),
            pltpu.VMEM((NIDX, 4 * RANK), jnp.float32),
            pltpu.VMEM((CH,), jnp.float32),
            pltpu.SemaphoreType.DMA,
            pltpu.SemaphoreType.DMA,
        ],
    )
    out = run(cx, cy, cz, table)
    return out.reshape(n_rays, n_samples, 1)
